# baseline (speedup 1.0000x reference)
import jax
import jax.numpy as jnp
from jax.experimental import pallas as pl
from jax.experimental.pallas import tpu as pltpu

D_MODEL = 1024
SEQ_BLOCK = 512


def _film_body(ids_ref, x_ref, g_ref, b_ref, o_ref):
    o_ref[...] = g_ref[...] * x_ref[...] + b_ref[...]


def kernel(x, condition_ids, gamma, beta):
    B, S, D = x.shape
    N = gamma.shape[0]
    ids = condition_ids.astype(jnp.int32)
    g3 = gamma.reshape(N, 1, D)
    b3 = beta.reshape(N, 1, D)
    grid = (B, S // SEQ_BLOCK)
    return pl.pallas_call(
        _film_body,
        grid_spec=pltpu.PrefetchScalarGridSpec(
            num_scalar_prefetch=1,
            grid=grid,
            in_specs=[
                pl.BlockSpec((1, SEQ_BLOCK, D), lambda b, s, ids: (b, s, 0)),
                pl.BlockSpec((1, 1, D), lambda b, s, ids: (ids[b], 0, 0)),
                pl.BlockSpec((1, 1, D), lambda b, s, ids: (ids[b], 0, 0)),
            ],
            out_specs=pl.BlockSpec((1, SEQ_BLOCK, D), lambda b, s, ids: (b, s, 0)),
        ),
        out_shape=jax.ShapeDtypeStruct((B, S, D), x.dtype),
    )(ids, x, g3, b3)


# speedup vs baseline: 1.1307x; 1.1307x over previous
import jax
import jax.numpy as jnp
from jax.experimental import pallas as pl
from jax.experimental.pallas import tpu as pltpu

D_MODEL = 1024
SEQ_BLOCK = 2048


def _film_body(ids_ref, x_ref, g_ref, b_ref, o_ref):
    o_ref[...] = g_ref[...] * x_ref[...] + b_ref[...]


def kernel(x, condition_ids, gamma, beta):
    B, S, D = x.shape
    N = gamma.shape[0]
    ids = condition_ids.astype(jnp.int32)
    g3 = gamma.reshape(N, 1, D)
    b3 = beta.reshape(N, 1, D)
    grid = (B, S // SEQ_BLOCK)
    return pl.pallas_call(
        _film_body,
        grid_spec=pltpu.PrefetchScalarGridSpec(
            num_scalar_prefetch=1,
            grid=grid,
            in_specs=[
                pl.BlockSpec((1, SEQ_BLOCK, D), lambda b, s, ids: (b, s, 0)),
                pl.BlockSpec((1, 1, D), lambda b, s, ids: (ids[b], 0, 0)),
                pl.BlockSpec((1, 1, D), lambda b, s, ids: (ids[b], 0, 0)),
            ],
            out_specs=pl.BlockSpec((1, SEQ_BLOCK, D), lambda b, s, ids: (b, s, 0)),
        ),
        out_shape=jax.ShapeDtypeStruct((B, S, D), x.dtype),
    )(ids, x, g3, b3)
